# baseline (speedup 1.0000x reference)
import jax
import jax.numpy as jnp
from jax.experimental import pallas as pl

V = 1000
N = 3


def _fill_kernel(alpha_ref, batch_ref, out_ref, tgt_ref):
    a0 = alpha_ref[0, 0]
    a1 = alpha_ref[0, 1]
    a2 = alpha_ref[0, 2]
    s_full = (a0 + a1 + a2) * (1.0 / V)
    s_last = (a0 + a1) * (1.0 / V)
    n_preds = out_ref.shape[1]
    j = jax.lax.broadcasted_iota(jnp.int32, (n_preds, V), 0)
    vals = jnp.where(j < n_preds - 1, s_full, s_last)
    out_ref[...] = jnp.broadcast_to(vals[None], out_ref.shape)
    tgt_ref[...] = batch_ref[:, N - 1:]


def kernel(batch, TEXT, alpha):
    B, bptt = batch.shape
    n_preds = bptt - (N - 1) + 1
    n_tgt = n_preds - 1
    alpha2d = alpha.reshape(1, 3)

    BB = 32
    grid = (B // BB,)

    outputs, targets = pl.pallas_call(
        _fill_kernel,
        grid=grid,
        in_specs=[
            pl.BlockSpec((1, 3), lambda i: (0, 0)),
            pl.BlockSpec((BB, bptt), lambda i: (i, 0)),
        ],
        out_specs=[
            pl.BlockSpec((BB, n_preds, V), lambda i: (i, 0, 0)),
            pl.BlockSpec((BB, n_tgt), lambda i: (i, 0)),
        ],
        out_shape=[
            jax.ShapeDtypeStruct((B, n_preds, V), jnp.float32),
            jax.ShapeDtypeStruct((B, n_tgt), batch.dtype),
        ],
    )(alpha2d, batch)
    return outputs, targets
